# baseline (speedup 1.0000x reference)
import functools

import jax
import jax.numpy as jnp
from jax import lax
from jax.experimental import pallas as pl
from jax.experimental.pallas import tpu as pltpu
from jax.experimental.pallas import tpu_sc as plsc

N_NODES = 10000
N_EDGES = 320000
D = 128

NC = 2
NS = 16
NW = NC * NS

CHUNK = 128
C0 = 115
C1 = 43
E_PAD = NS * (C0 + C1) * CHUNK
AGG_ROWS = 10008
STRIPE = 624
LAST_STRIPE = AGG_ROWS - 15 * STRIPE


def _sc_segment_sum(x, src0, src1, dst0, dst1, zeros):
  mesh = plsc.VectorSubcoreMesh(core_axis_name="c", subcore_axis_name="s")

  @functools.partial(
      pl.kernel,
      out_type=jax.ShapeDtypeStruct((NC, AGG_ROWS, D), jnp.float32),
      mesh=mesh,
      scratch_types=[
          pltpu.VMEM((C0, CHUNK), jnp.int32),
          pltpu.VMEM((C0, CHUNK), jnp.int32),
          pltpu.VMEM((CHUNK, D), jnp.float32),
          pltpu.VMEM_SHARED((AGG_ROWS, D), jnp.float32),
          pltpu.SemaphoreType.DMA,
      ],
  )
  def k(x_hbm, src0_hbm, src1_hbm, dst0_hbm, dst1_hbm, zeros_hbm, out_hbm,
        src_v, dst_v, rows_v, agg_sh, g0):
    cid = lax.axis_index("c")
    sid = lax.axis_index("s")

    @pl.when(sid < NS - 1)
    def _():
      pltpu.sync_copy(zeros_hbm.at[pl.ds(sid * STRIPE, STRIPE)],
                      agg_sh.at[pl.ds(sid * STRIPE, STRIPE)])

    @pl.when(sid == NS - 1)
    def _():
      pltpu.sync_copy(zeros_hbm.at[pl.ds(15 * STRIPE, LAST_STRIPE)],
                      agg_sh.at[pl.ds(15 * STRIPE, LAST_STRIPE)])

    @pl.when(cid == 0)
    def _():
      pltpu.sync_copy(src0_hbm.at[sid], src_v)
      pltpu.sync_copy(dst0_hbm.at[sid], dst_v)

    @pl.when(cid == 1)
    def _():
      pltpu.sync_copy(src1_hbm.at[sid], src_v.at[pl.ds(0, C1)])
      pltpu.sync_copy(dst1_hbm.at[sid], dst_v.at[pl.ds(0, C1)])

    plsc.subcore_barrier()

    def body(j, carry):
      pltpu.async_copy(x_hbm.at[src_v.at[j]], rows_v, g0).wait()
      pltpu.sync_copy(rows_v, agg_sh.at[dst_v.at[j]], add=True)
      return carry

    @pl.when(cid == 0)
    def _():
      lax.fori_loop(0, C0, body, 0, unroll=3)

    @pl.when(cid == 1)
    def _():
      lax.fori_loop(0, C1, body, 0, unroll=1)

    plsc.subcore_barrier()

    @pl.when(sid < NS - 1)
    def _():
      pltpu.sync_copy(agg_sh.at[pl.ds(sid * STRIPE, STRIPE)],
                      out_hbm.at[cid, pl.ds(sid * STRIPE, STRIPE)])

    @pl.when(sid == NS - 1)
    def _():
      pltpu.sync_copy(agg_sh.at[pl.ds(15 * STRIPE, LAST_STRIPE)],
                      out_hbm.at[cid, pl.ds(15 * STRIPE, LAST_STRIPE)])

  return k(x, src0, src1, dst0, dst1, zeros)


def _tc_linear(x, p, w1t, w2t, b2):
  blk = 1000

  def body(x_ref, p_ref, w1_ref, w2_ref, b_ref, o_ref):
    agg = p_ref[0] + p_ref[1]
    o_ref[...] = (
        jnp.dot(x_ref[...], w1_ref[...], preferred_element_type=jnp.float32)
        + jnp.dot(agg, w2_ref[...], preferred_element_type=jnp.float32)
        + b_ref[...]
    )

  return pl.pallas_call(
      body,
      grid=(N_NODES // blk,),
      in_specs=[
          pl.BlockSpec((blk, D), lambda i: (i, 0)),
          pl.BlockSpec((NC, blk, D), lambda i: (0, i, 0)),
          pl.BlockSpec((D, D), lambda i: (0, 0)),
          pl.BlockSpec((D, D), lambda i: (0, 0)),
          pl.BlockSpec((1, D), lambda i: (0, 0)),
      ],
      out_specs=pl.BlockSpec((blk, D), lambda i: (i, 0)),
      out_shape=jax.ShapeDtypeStruct((N_NODES, D), jnp.float32),
  )(x, p, w1t, w2t, b2)


@jax.jit
def kernel(x, edge_index, W, b):
  e0 = NS * C0 * CHUNK
  pad = E_PAD - N_EDGES
  src0 = edge_index[0, :e0].reshape(NS, C0, CHUNK)
  dst0 = edge_index[1, :e0].reshape(NS, C0, CHUNK)
  src1 = jnp.concatenate(
      [edge_index[0, e0:], jnp.zeros((pad,), jnp.int32)]
  ).reshape(NS, C1, CHUNK)
  dst1 = jnp.concatenate(
      [edge_index[1, e0:], jnp.full((pad,), N_NODES, jnp.int32)]
  ).reshape(NS, C1, CHUNK)
  zeros = jnp.zeros((AGG_ROWS, D), jnp.float32)
  p = _sc_segment_sum(x, src0, src1, dst0, dst1, zeros)
  w1t = W[:, :D].T
  w2t = W[:, D:].T
  return _tc_linear(x, p, w1t, w2t, b.reshape(1, D))


# speedup vs baseline: 1.0124x; 1.0124x over previous
import functools

import jax
import jax.numpy as jnp
from jax import lax
from jax.experimental import pallas as pl
from jax.experimental.pallas import tpu as pltpu
from jax.experimental.pallas import tpu_sc as plsc

N_NODES = 10000
N_EDGES = 320000
D = 128

NC = 2
NS = 16
NW = NC * NS

CHUNK = 128
C0 = 117
C1 = 41
E_PAD = NS * (C0 + C1) * CHUNK
AGG_ROWS = 10008
STRIPE = 624
LAST_STRIPE = AGG_ROWS - 15 * STRIPE


def _sc_segment_sum(x, src0, src1, dst0, dst1, zeros):
  mesh = plsc.VectorSubcoreMesh(core_axis_name="c", subcore_axis_name="s")

  @functools.partial(
      pl.kernel,
      out_type=jax.ShapeDtypeStruct((NC, AGG_ROWS, D), jnp.float32),
      mesh=mesh,
      scratch_types=[
          pltpu.VMEM((C0, CHUNK), jnp.int32),
          pltpu.VMEM((C0, CHUNK), jnp.int32),
          pltpu.VMEM((CHUNK, D), jnp.float32),
          pltpu.VMEM_SHARED((AGG_ROWS, D), jnp.float32),
          pltpu.SemaphoreType.DMA,
      ],
  )
  def k(x_hbm, src0_hbm, src1_hbm, dst0_hbm, dst1_hbm, zeros_hbm, out_hbm,
        src_v, dst_v, rows_v, agg_sh, g0):
    cid = lax.axis_index("c")
    sid = lax.axis_index("s")

    @pl.when(sid < NS - 1)
    def _():
      pltpu.sync_copy(zeros_hbm.at[pl.ds(sid * STRIPE, STRIPE)],
                      agg_sh.at[pl.ds(sid * STRIPE, STRIPE)])

    @pl.when(sid == NS - 1)
    def _():
      pltpu.sync_copy(zeros_hbm.at[pl.ds(15 * STRIPE, LAST_STRIPE)],
                      agg_sh.at[pl.ds(15 * STRIPE, LAST_STRIPE)])

    @pl.when(cid == 0)
    def _():
      pltpu.sync_copy(src0_hbm.at[sid], src_v)
      pltpu.sync_copy(dst0_hbm.at[sid], dst_v)

    @pl.when(cid == 1)
    def _():
      pltpu.sync_copy(src1_hbm.at[sid], src_v.at[pl.ds(0, C1)])
      pltpu.sync_copy(dst1_hbm.at[sid], dst_v.at[pl.ds(0, C1)])

    plsc.subcore_barrier()

    def body(j, carry):
      pltpu.async_copy(x_hbm.at[src_v.at[j]], rows_v, g0).wait()
      pltpu.sync_copy(rows_v, agg_sh.at[dst_v.at[j]], add=True)
      return carry

    @pl.when(cid == 0)
    def _():
      lax.fori_loop(0, C0, body, 0, unroll=3)

    @pl.when(cid == 1)
    def _():
      lax.fori_loop(0, C1, body, 0, unroll=1)

    plsc.subcore_barrier()

    @pl.when(sid < NS - 1)
    def _():
      pltpu.sync_copy(agg_sh.at[pl.ds(sid * STRIPE, STRIPE)],
                      out_hbm.at[cid, pl.ds(sid * STRIPE, STRIPE)])

    @pl.when(sid == NS - 1)
    def _():
      pltpu.sync_copy(agg_sh.at[pl.ds(15 * STRIPE, LAST_STRIPE)],
                      out_hbm.at[cid, pl.ds(15 * STRIPE, LAST_STRIPE)])

  return k(x, src0, src1, dst0, dst1, zeros)


def _tc_linear(x, p, w1t, w2t, b2):
  blk = 1000

  def body(x_ref, p_ref, w1_ref, w2_ref, b_ref, o_ref):
    agg = p_ref[0] + p_ref[1]
    o_ref[...] = (
        jnp.dot(x_ref[...], w1_ref[...], preferred_element_type=jnp.float32)
        + jnp.dot(agg, w2_ref[...], preferred_element_type=jnp.float32)
        + b_ref[...]
    )

  return pl.pallas_call(
      body,
      grid=(N_NODES // blk,),
      in_specs=[
          pl.BlockSpec((blk, D), lambda i: (i, 0)),
          pl.BlockSpec((NC, blk, D), lambda i: (0, i, 0)),
          pl.BlockSpec((D, D), lambda i: (0, 0)),
          pl.BlockSpec((D, D), lambda i: (0, 0)),
          pl.BlockSpec((1, D), lambda i: (0, 0)),
      ],
      out_specs=pl.BlockSpec((blk, D), lambda i: (i, 0)),
      out_shape=jax.ShapeDtypeStruct((N_NODES, D), jnp.float32),
  )(x, p, w1t, w2t, b2)


@jax.jit
def kernel(x, edge_index, W, b):
  e0 = NS * C0 * CHUNK
  pad = E_PAD - N_EDGES
  src0 = edge_index[0, :e0].reshape(NS, C0, CHUNK)
  dst0 = edge_index[1, :e0].reshape(NS, C0, CHUNK)
  src1 = jnp.concatenate(
      [edge_index[0, e0:], jnp.zeros((pad,), jnp.int32)]
  ).reshape(NS, C1, CHUNK)
  dst1 = jnp.concatenate(
      [edge_index[1, e0:], jnp.full((pad,), N_NODES, jnp.int32)]
  ).reshape(NS, C1, CHUNK)
  zeros = jnp.zeros((AGG_ROWS, D), jnp.float32)
  p = _sc_segment_sum(x, src0, src1, dst0, dst1, zeros)
  w1t = W[:, :D].T
  w2t = W[:, D:].T
  return _tc_linear(x, p, w1t, w2t, b.reshape(1, D))


# speedup vs baseline: 1.0475x; 1.0347x over previous
import functools

import jax
import jax.numpy as jnp
from jax import lax
from jax.experimental import pallas as pl
from jax.experimental.pallas import tpu as pltpu
from jax.experimental.pallas import tpu_sc as plsc

N_NODES = 10000
N_EDGES = 320000
D = 128

NC = 2
NS = 16
NW = NC * NS

CHUNK = 128
C0 = 117
C1 = 41
E_PAD = NS * (C0 + C1) * CHUNK
AGG_ROWS = 10008
STRIPE = 624
LAST_STRIPE = AGG_ROWS - 15 * STRIPE


def _sc_segment_sum(x, ei0, ei1, zeros):
  mesh = plsc.VectorSubcoreMesh(core_axis_name="c", subcore_axis_name="s")

  @functools.partial(
      pl.kernel,
      out_type=jax.ShapeDtypeStruct((NC, AGG_ROWS, D), jnp.float32),
      mesh=mesh,
      scratch_types=[
          pltpu.VMEM((C0, CHUNK), jnp.int32),
          pltpu.VMEM((C0, CHUNK), jnp.int32),
          pltpu.VMEM((CHUNK, D), jnp.float32),
          pltpu.VMEM_SHARED((AGG_ROWS, D), jnp.float32),
          pltpu.SemaphoreType.DMA,
      ],
  )
  def k(x_hbm, ei0_hbm, ei1_hbm, zeros_hbm, out_hbm,
        src_v, dst_v, rows_v, agg_sh, g0):
    cid = lax.axis_index("c")
    sid = lax.axis_index("s")

    @pl.when(sid < NS - 1)
    def _():
      pltpu.sync_copy(zeros_hbm.at[pl.ds(sid * STRIPE, STRIPE)],
                      agg_sh.at[pl.ds(sid * STRIPE, STRIPE)])

    @pl.when(sid == NS - 1)
    def _():
      pltpu.sync_copy(zeros_hbm.at[pl.ds(15 * STRIPE, LAST_STRIPE)],
                      agg_sh.at[pl.ds(15 * STRIPE, LAST_STRIPE)])

    @pl.when(cid == 0)
    def _():
      pltpu.sync_copy(ei0_hbm.at[0, sid], src_v)
      pltpu.sync_copy(ei0_hbm.at[1, sid], dst_v)

    @pl.when(cid == 1)
    def _():
      pltpu.sync_copy(ei1_hbm.at[0, sid], src_v.at[pl.ds(0, C1)])
      pltpu.sync_copy(ei1_hbm.at[1, sid], dst_v.at[pl.ds(0, C1)])

    plsc.subcore_barrier()

    def body(j, carry):
      pltpu.async_copy(x_hbm.at[src_v.at[j]], rows_v, g0).wait()
      pltpu.sync_copy(rows_v, agg_sh.at[dst_v.at[j]], add=True)
      return carry

    @pl.when(cid == 0)
    def _():
      lax.fori_loop(0, C0, body, 0, unroll=3)

    @pl.when(cid == 1)
    def _():
      lax.fori_loop(0, C1, body, 0, unroll=1)

    plsc.subcore_barrier()

    @pl.when(sid < NS - 1)
    def _():
      pltpu.sync_copy(agg_sh.at[pl.ds(sid * STRIPE, STRIPE)],
                      out_hbm.at[cid, pl.ds(sid * STRIPE, STRIPE)])

    @pl.when(sid == NS - 1)
    def _():
      pltpu.sync_copy(agg_sh.at[pl.ds(15 * STRIPE, LAST_STRIPE)],
                      out_hbm.at[cid, pl.ds(15 * STRIPE, LAST_STRIPE)])

  return k(x, ei0, ei1, zeros)


def _tc_linear(x, p, w1t, w2t, b2):
  blk = 1000

  def body(x_ref, p_ref, w1_ref, w2_ref, b_ref, o_ref):
    agg = p_ref[0] + p_ref[1]
    o_ref[...] = (
        jnp.dot(x_ref[...], w1_ref[...], preferred_element_type=jnp.float32)
        + jnp.dot(agg, w2_ref[...], preferred_element_type=jnp.float32)
        + b_ref[...]
    )

  return pl.pallas_call(
      body,
      grid=(N_NODES // blk,),
      in_specs=[
          pl.BlockSpec((blk, D), lambda i: (i, 0)),
          pl.BlockSpec((NC, blk, D), lambda i: (0, i, 0)),
          pl.BlockSpec((D, D), lambda i: (0, 0)),
          pl.BlockSpec((D, D), lambda i: (0, 0)),
          pl.BlockSpec((1, D), lambda i: (0, 0)),
      ],
      out_specs=pl.BlockSpec((blk, D), lambda i: (i, 0)),
      out_shape=jax.ShapeDtypeStruct((N_NODES, D), jnp.float32),
  )(x, p, w1t, w2t, b2)


@jax.jit
def kernel(x, edge_index, W, b):
  e0 = NS * C0 * CHUNK
  pad = E_PAD - N_EDGES
  ei0 = edge_index[:, :e0].reshape(2, NS, C0, CHUNK)
  padv = jnp.stack([jnp.zeros((pad,), jnp.int32),
                    jnp.full((pad,), N_NODES, jnp.int32)])
  ei1 = jnp.concatenate([edge_index[:, e0:], padv],
                        axis=1).reshape(2, NS, C1, CHUNK)
  zeros = jnp.zeros((AGG_ROWS, D), jnp.float32)
  p = _sc_segment_sum(x, ei0, ei1, zeros)
  w1t = W[:, :D].T
  w2t = W[:, D:].T
  return _tc_linear(x, p, w1t, w2t, b.reshape(1, D))
